# Initial kernel scaffold; baseline (speedup 1.0000x reference)
import jax
import jax.numpy as jnp
from jax.experimental import pallas as pl


def kernel(indices, table):
    raise NotImplementedError("write your pallas kernel here")


# speedup vs baseline: 4.2430x; 4.2430x over previous
import functools

import jax
import jax.numpy as jnp
from jax import lax
from jax.experimental import pallas as pl
from jax.experimental.pallas import tpu as pltpu
from jax.experimental.pallas import tpu_sc as plsc

VOCAB = 100000
D = 64
SEQ = 200
BATCH = 4096

NUM_WORKERS = 32
TOTAL = BATCH * SEQ
PER_W = TOTAL // NUM_WORKERS
CHUNK = 400
NCHUNK = PER_W // CHUNK
NBUF = 3
LANES = 16


def _positional_encoding():
    even_i = jnp.arange(0, D, 2).astype(jnp.float32)
    denominator = jnp.power(10000.0, even_i / D)
    position = jnp.arange(SEQ).reshape(SEQ, 1).astype(jnp.float32)
    even_pe = jnp.sin(position / denominator)
    odd_pe = jnp.cos(position / denominator)
    return jnp.stack([even_pe, odd_pe], axis=2).reshape(SEQ, D)


_mesh = plsc.VectorSubcoreMesh(core_axis_name="c", subcore_axis_name="s")


@functools.partial(
    pl.kernel,
    out_type=jax.ShapeDtypeStruct((TOTAL, D), jnp.float32),
    mesh=_mesh,
    compiler_params=pltpu.CompilerParams(use_tc_tiling_on_sc=False),
    scratch_types=[
        pltpu.VMEM((PER_W,), jnp.int32),
        pltpu.VMEM((SEQ, D), jnp.float32),
        pltpu.VMEM((CHUNK, D), jnp.float32),
        pltpu.VMEM((CHUNK, D), jnp.float32),
        pltpu.VMEM((CHUNK, D), jnp.float32),
        pltpu.SemaphoreType.DMA,
        pltpu.SemaphoreType.DMA,
        pltpu.SemaphoreType.DMA,
        pltpu.SemaphoreType.DMA,
        pltpu.SemaphoreType.DMA,
        pltpu.SemaphoreType.DMA,
    ],
)
def _embed(table_hbm, idx_hbm, pe_hbm, out_hbm,
           idx_v, pe_v, rows0, rows1, rows2,
           g0, g1, g2, s0, s1, s2):
    rows = (rows0, rows1, rows2)
    gsems = (g0, g1, g2)
    ssems = (s0, s1, s2)

    wid = lax.axis_index("s") * 2 + lax.axis_index("c")
    base = wid * PER_W

    pltpu.sync_copy(idx_hbm.at[pl.ds(base, PER_W)], idx_v)
    pltpu.sync_copy(pe_hbm, pe_v)

    def gather_desc(g, b):
        return pltpu.make_async_copy(
            table_hbm.at[idx_v.at[pl.ds(g * CHUNK, CHUNK)]],
            rows[b], gsems[b])

    def scatter_desc(g, b):
        return pltpu.make_async_copy(
            rows[b], out_hbm.at[pl.ds(base + g * CHUNK, CHUNK)], ssems[b])

    def add_pe(b):
        rbuf = rows[b]
        for rep in range(CHUNK // SEQ):
            @pl.loop(0, SEQ)
            def _(r):
                row = rep * SEQ + r
                for c in range(D // LANES):
                    sl = pl.ds(c * LANES, LANES)
                    rbuf[row, sl] = rbuf[row, sl] + pe_v[r, sl]

    def process(g, b):
        gather_desc(g, b).wait()
        add_pe(b)
        scatter_desc(g, b).start()
        nb = (b + 2) % NBUF

        @pl.when(g + 2 < NCHUNK)
        def _():
            @pl.when(g >= 1)
            def _():
                scatter_desc(g - 1, nb).wait()
            gather_desc(g + 2, nb).start()

    gather_desc(0, 0).start()
    gather_desc(1, 1).start()

    @pl.loop(0, NCHUNK - 1, step=NBUF)
    def _(go):
        for b in range(NBUF):
            process(go + b, b)

    process(NCHUNK - 1, 0)

    for g in (NCHUNK - 3, NCHUNK - 2, NCHUNK - 1):
        scatter_desc(g, g % NBUF).wait()


def kernel(indices, table):
    idx_flat = indices.reshape(TOTAL).astype(jnp.int32)
    pe = _positional_encoding()
    out = _embed(table, idx_flat, pe)
    return out.reshape(BATCH, SEQ, D)
